# Initial kernel scaffold; baseline (speedup 1.0000x reference)
import jax
import jax.numpy as jnp
from jax.experimental import pallas as pl


def kernel(x, edge_index, W, b):
    raise NotImplementedError("write your pallas kernel here")


# speedup vs baseline: 22.3187x; 22.3187x over previous
import dataclasses
import functools

import jax
import jax.numpy as jnp
from jax import lax
from jax.experimental import pallas as pl
from jax.experimental.pallas import tpu as pltpu
from jax.experimental.pallas import tpu_sc as plsc

NC = 2
NS = 16
NW = NC * NS
L = 16

K = 128


def _sc_compiler_params():
    cp = pltpu.CompilerParams()
    fields = pltpu.CompilerParams.__dataclass_fields__
    if "needs_layout_passes" in fields:
        cp = dataclasses.replace(cp, needs_layout_passes=False)
    if "use_tc_tiling_on_sc" in fields:
        cp = dataclasses.replace(cp, use_tc_tiling_on_sc=False)
    return cp


def _round_up(a, m):
    return (a + m - 1) // m * m


def _matmul(x, W):
    n, _ = x.shape
    f_out = W.shape[1]

    def body(x_ref, w_ref, o_ref):
        o_ref[...] = jnp.dot(x_ref[...], w_ref[...],
                             preferred_element_type=jnp.float32)

    return pl.pallas_call(
        body,
        out_shape=jax.ShapeDtypeStruct((n, f_out), jnp.float32),
    )(x, W)


def _scale(h, degp_t):
    n, f_out = h.shape

    def body(h_ref, d_ref, g_ref, s_ref):
        deg = d_ref[:, 0:1] + d_ref[:, 1:2] + 1.0
        dis = lax.rsqrt(deg)
        g_ref[...] = dis * h_ref[...]
        s_ref[...] = dis / deg

    return pl.pallas_call(
        body,
        out_shape=[
            jax.ShapeDtypeStruct((n, f_out), jnp.float32),
            jax.ShapeDtypeStruct((n, 1), jnp.float32),
        ],
    )(h, degp_t)


def _finish(acc, g, dscale, b2):
    n, f_out = g.shape

    def body(acc_ref, g_ref, s_ref, b_ref, o_ref):
        tot = acc_ref[0, :n, :] + acc_ref[1, :n, :] + g_ref[...]
        y = s_ref[...] * tot + b_ref[...]
        m = jnp.max(y, axis=1, keepdims=True)
        z = y - m
        o_ref[...] = z - jnp.log(jnp.sum(jnp.exp(z), axis=1, keepdims=True))

    return pl.pallas_call(
        body,
        out_shape=jax.ShapeDtypeStruct((n, f_out), jnp.float32),
    )(acc, g, dscale, b2)


def _degrees(dst_pad, n_pad, e_pad):
    ew = e_pad // NW
    rpw = n_pad // NS
    mesh = plsc.VectorSubcoreMesh(core_axis_name="c", subcore_axis_name="s")

    @functools.partial(
        pl.kernel,
        out_type=jax.ShapeDtypeStruct((NC, n_pad), jnp.float32),
        mesh=mesh,
        scratch_types=[
            pltpu.VMEM((ew,), jnp.int32),
            pltpu.VMEM((n_pad,), jnp.float32),
            pltpu.VMEM((rpw,), jnp.float32),
            pltpu.VMEM((rpw,), jnp.float32),
            pltpu.VMEM_SHARED((NS, n_pad), jnp.float32),
        ],
        compiler_params=_sc_compiler_params(),
    )
    def kern(dst_hbm, out_hbm, idx_v, cnt_v, acc_v, tmp_v, part_sh):
        c = lax.axis_index("c")
        s = lax.axis_index("s")
        w = c * NS + s

        zero16 = jnp.zeros((L,), jnp.float32)
        ones16 = jnp.ones((L,), jnp.float32)

        @pl.loop(0, n_pad, step=L)
        def _(i):
            cnt_v[pl.ds(i, L)] = zero16

        pltpu.sync_copy(dst_hbm.at[pl.ds(w * ew, ew)], idx_v)

        @pl.loop(0, ew, step=L)
        def _(i):
            plsc.addupdate_scatter(cnt_v, [idx_v[pl.ds(i, L)]], ones16)

        pltpu.sync_copy(cnt_v, part_sh.at[s])
        plsc.subcore_barrier()

        base = s * rpw
        pltpu.sync_copy(part_sh.at[0, pl.ds(base, rpw)], acc_v)

        @pl.loop(1, NS)
        def _(j):
            pltpu.sync_copy(part_sh.at[j, pl.ds(base, rpw)], tmp_v)

            @pl.loop(0, rpw, step=L)
            def _(k):
                acc_v[pl.ds(k, L)] = acc_v[pl.ds(k, L)] + tmp_v[pl.ds(k, L)]

        pltpu.sync_copy(acc_v, out_hbm.at[c, pl.ds(base, rpw)])

    return kern(dst_pad)


def _message_pass(g, src_pad, dst_pad, n_pad, e_pad):
    f_out = g.shape[1]
    ew = e_pad // NW
    nch = ew // K
    rpw = n_pad // NS
    zr = 64
    mesh = plsc.VectorSubcoreMesh(core_axis_name="c", subcore_axis_name="s")

    @functools.partial(
        pl.kernel,
        out_type=jax.ShapeDtypeStruct((NC, n_pad, f_out), jnp.float32),
        mesh=mesh,
        scratch_types=[
            pltpu.VMEM((K,), jnp.int32),
            pltpu.VMEM((K,), jnp.int32),
            pltpu.VMEM((K, f_out), jnp.float32),
            pltpu.VMEM((zr, f_out), jnp.float32),
            pltpu.VMEM_SHARED((n_pad, f_out), jnp.float32),
        ],
        compiler_params=_sc_compiler_params(),
    )
    def kern(src_hbm, dst_hbm, g_hbm, out_hbm, sidx_v, didx_v, rows_v,
             zer_v, acc_sh):
        c = lax.axis_index("c")
        s = lax.axis_index("s")
        w = c * NS + s

        zero16 = jnp.zeros((L,), jnp.float32)

        @pl.loop(0, zr)
        def _(r):
            @pl.loop(0, f_out, step=L)
            def _(j):
                zer_v[r, pl.ds(j, L)] = zero16

        base_row = s * rpw

        @pl.loop(0, rpw, step=zr)
        def _(r):
            pltpu.sync_copy(zer_v, acc_sh.at[pl.ds(base_row + r, zr)])

        plsc.subcore_barrier()

        @pl.loop(0, nch)
        def _(i):
            e0 = w * ew + i * K
            pltpu.sync_copy(src_hbm.at[pl.ds(e0, K)], sidx_v)
            pltpu.sync_copy(dst_hbm.at[pl.ds(e0, K)], didx_v)
            pltpu.sync_copy(g_hbm.at[sidx_v], rows_v)
            pltpu.sync_copy(rows_v, acc_sh.at[didx_v], add=True)

        plsc.subcore_barrier()
        pltpu.sync_copy(acc_sh.at[pl.ds(base_row, rpw)],
                        out_hbm.at[c, pl.ds(base_row, rpw)])

    return kern(src_pad, dst_pad, g)


def kernel(x, edge_index, W, b):
    n, _ = x.shape
    f_out = W.shape[1]
    e = edge_index.shape[1]

    n_pad = _round_up(n + 1, NS * 64)
    e_pad = _round_up(e, NW * K)

    pad_e = e_pad - e
    src_pad = jnp.concatenate(
        [edge_index[0], jnp.zeros((pad_e,), jnp.int32)])
    dst_pad = jnp.concatenate(
        [edge_index[1], jnp.full((pad_e,), n, jnp.int32)])

    h = _matmul(x, W)
    degp = _degrees(dst_pad, n_pad, e_pad)

    degp_t = jnp.transpose(degp)[:n]
    g, dscale = _scale(h, degp_t)

    acc = _message_pass(g, src_pad, dst_pad, n_pad, e_pad)

    return _finish(acc, g, dscale, jnp.reshape(b, (1, f_out)))


# speedup vs baseline: 24.2089x; 1.0847x over previous
import dataclasses
import functools

import jax
import jax.numpy as jnp
from jax import lax
from jax.experimental import pallas as pl
from jax.experimental.pallas import tpu as pltpu
from jax.experimental.pallas import tpu_sc as plsc

NC = 2
NS = 16
NW = NC * NS
L = 16

K = 128


def _sc_compiler_params():
    cp = pltpu.CompilerParams()
    fields = pltpu.CompilerParams.__dataclass_fields__
    if "needs_layout_passes" in fields:
        cp = dataclasses.replace(cp, needs_layout_passes=False)
    if "use_tc_tiling_on_sc" in fields:
        cp = dataclasses.replace(cp, use_tc_tiling_on_sc=False)
    return cp


def _round_up(a, m):
    return (a + m - 1) // m * m


def _matmul(x, W):
    n, _ = x.shape
    f_out = W.shape[1]

    def body(x_ref, w_ref, o_ref):
        o_ref[...] = jnp.dot(x_ref[...], w_ref[...],
                             preferred_element_type=jnp.float32)

    return pl.pallas_call(
        body,
        out_shape=jax.ShapeDtypeStruct((n, f_out), jnp.float32),
    )(x, W)


def _scale(h, degp_t):
    n, f_out = h.shape

    def body(h_ref, d_ref, g_ref, s_ref):
        deg = d_ref[:, 0:1] + d_ref[:, 1:2] + 1.0
        dis = lax.rsqrt(deg)
        g_ref[...] = dis * h_ref[...]
        s_ref[...] = dis / deg

    return pl.pallas_call(
        body,
        out_shape=[
            jax.ShapeDtypeStruct((n, f_out), jnp.float32),
            jax.ShapeDtypeStruct((n, 1), jnp.float32),
        ],
    )(h, degp_t)


def _finish(acc, g, dscale, b2):
    n, f_out = g.shape

    def body(acc_ref, g_ref, s_ref, b_ref, o_ref):
        tot = acc_ref[0, :n, :] + acc_ref[1, :n, :] + g_ref[...]
        y = s_ref[...] * tot + b_ref[...]
        m = jnp.max(y, axis=1, keepdims=True)
        z = y - m
        o_ref[...] = z - jnp.log(jnp.sum(jnp.exp(z), axis=1, keepdims=True))

    return pl.pallas_call(
        body,
        out_shape=jax.ShapeDtypeStruct((n, f_out), jnp.float32),
    )(acc, g, dscale, b2)


def _degrees(dst_pad, n_pad, e_pad):
    ew = e_pad // NW
    rpw = n_pad // NS
    mesh = plsc.VectorSubcoreMesh(core_axis_name="c", subcore_axis_name="s")

    @functools.partial(
        pl.kernel,
        out_type=jax.ShapeDtypeStruct((NC, n_pad), jnp.float32),
        mesh=mesh,
        scratch_types=[
            pltpu.VMEM((ew,), jnp.int32),
            pltpu.VMEM((n_pad,), jnp.float32),
            pltpu.VMEM((rpw,), jnp.float32),
            pltpu.VMEM((rpw,), jnp.float32),
            pltpu.VMEM_SHARED((NS, n_pad), jnp.float32),
        ],
        compiler_params=_sc_compiler_params(),
    )
    def kern(dst_hbm, out_hbm, idx_v, cnt_v, acc_v, tmp_v, part_sh):
        c = lax.axis_index("c")
        s = lax.axis_index("s")
        w = c * NS + s

        zero16 = jnp.zeros((L,), jnp.float32)
        ones16 = jnp.ones((L,), jnp.float32)

        @pl.loop(0, n_pad, step=L)
        def _(i):
            cnt_v[pl.ds(i, L)] = zero16

        pltpu.sync_copy(dst_hbm.at[pl.ds(w * ew, ew)], idx_v)

        @pl.loop(0, ew, step=L)
        def _(i):
            plsc.addupdate_scatter(cnt_v, [idx_v[pl.ds(i, L)]], ones16)

        pltpu.sync_copy(cnt_v, part_sh.at[s])
        plsc.subcore_barrier()

        base = s * rpw
        pltpu.sync_copy(part_sh.at[0, pl.ds(base, rpw)], acc_v)

        @pl.loop(1, NS)
        def _(j):
            pltpu.sync_copy(part_sh.at[j, pl.ds(base, rpw)], tmp_v)

            @pl.loop(0, rpw, step=L)
            def _(k):
                acc_v[pl.ds(k, L)] = acc_v[pl.ds(k, L)] + tmp_v[pl.ds(k, L)]

        pltpu.sync_copy(acc_v, out_hbm.at[c, pl.ds(base, rpw)])

    return kern(dst_pad)


def _message_pass(g, src_r, dst_r, n_pad, e_pad):
    f_out = g.shape[1]
    nch = src_r.shape[1]
    rpw = n_pad // NS
    zr = 64
    mesh = plsc.VectorSubcoreMesh(core_axis_name="c", subcore_axis_name="s")

    @functools.partial(
        pl.kernel,
        out_type=jax.ShapeDtypeStruct((NC, n_pad, f_out), jnp.float32),
        mesh=mesh,
        scratch_types=[
            pltpu.VMEM((nch, 1, K), jnp.int32),
            pltpu.VMEM((nch, 1, K), jnp.int32),
            pltpu.VMEM((K, f_out), jnp.float32),
            pltpu.VMEM((K, f_out), jnp.float32),
            pltpu.VMEM((zr, f_out), jnp.float32),
            pltpu.VMEM_SHARED((n_pad, f_out), jnp.float32),
            pltpu.SemaphoreType.DMA,
            pltpu.SemaphoreType.DMA,
        ],
        compiler_params=_sc_compiler_params(),
    )
    def kern(src_hbm, dst_hbm, g_hbm, out_hbm, sidx_v, didx_v, rows_a,
             rows_b, zer_v, acc_sh, sem_a, sem_b):
        c = lax.axis_index("c")
        s = lax.axis_index("s")
        w = c * NS + s

        zero16 = jnp.zeros((L,), jnp.float32)

        @pl.loop(0, zr)
        def _(r):
            @pl.loop(0, f_out, step=L)
            def _(j):
                zer_v[r, pl.ds(j, L)] = zero16

        pltpu.sync_copy(src_hbm.at[w], sidx_v)
        pltpu.sync_copy(dst_hbm.at[w], didx_v)

        base_row = s * rpw

        @pl.loop(0, rpw, step=zr)
        def _(r):
            pltpu.sync_copy(zer_v, acc_sh.at[pl.ds(base_row + r, zr)])

        plsc.subcore_barrier()

        pltpu.async_copy(g_hbm.at[sidx_v.at[0, 0]], rows_a, sem_a)
        pltpu.async_copy(g_hbm.at[sidx_v.at[1, 0]], rows_b, sem_b)

        @pl.loop(0, nch, step=2)
        def _(i):
            pltpu.make_async_copy(g_hbm.at[sidx_v.at[i, 0]], rows_a,
                                  sem_a).wait()
            pltpu.sync_copy(rows_a, acc_sh.at[didx_v.at[i, 0]], add=True)

            @pl.when(i + 2 < nch)
            def _():
                pltpu.async_copy(g_hbm.at[sidx_v.at[i + 2, 0]], rows_a,
                                 sem_a)

            pltpu.make_async_copy(g_hbm.at[sidx_v.at[i + 1, 0]], rows_b,
                                  sem_b).wait()
            pltpu.sync_copy(rows_b, acc_sh.at[didx_v.at[i + 1, 0]], add=True)

            @pl.when(i + 3 < nch)
            def _():
                pltpu.async_copy(g_hbm.at[sidx_v.at[i + 3, 0]], rows_b,
                                 sem_b)

        plsc.subcore_barrier()
        pltpu.sync_copy(acc_sh.at[pl.ds(base_row, rpw)],
                        out_hbm.at[c, pl.ds(base_row, rpw)])

    return kern(src_r, dst_r, g)


def kernel(x, edge_index, W, b):
    n, _ = x.shape
    f_out = W.shape[1]
    e = edge_index.shape[1]

    n_pad = _round_up(n + 1, NS * 64)
    e_pad = _round_up(e, NW * K * 2)

    pad_e = e_pad - e
    src_pad = jnp.concatenate(
        [edge_index[0], jnp.zeros((pad_e,), jnp.int32)])
    dst_pad = jnp.concatenate(
        [edge_index[1], jnp.full((pad_e,), n, jnp.int32)])
    nch = e_pad // (NW * K)
    src_r = jnp.reshape(src_pad, (NW, nch, 1, K))
    dst_r = jnp.reshape(dst_pad, (NW, nch, 1, K))

    h = _matmul(x, W)
    degp = _degrees(dst_pad, n_pad, e_pad)

    degp_t = jnp.transpose(degp)[:n]
    g, dscale = _scale(h, degp_t)

    acc = _message_pass(g, src_r, dst_r, n_pad, e_pad)

    return _finish(acc, g, dscale, jnp.reshape(b, (1, f_out)))
